# Initial kernel scaffold; baseline (speedup 1.0000x reference)
import jax
import jax.numpy as jnp
from jax.experimental import pallas as pl


def kernel(points, params):
    raise NotImplementedError("write your pallas kernel here")


# speedup vs baseline: 26.0603x; 26.0603x over previous
import functools
import math

import jax
import jax.numpy as jnp
import numpy as np
from jax.experimental import pallas as pl

_BN_EPS = 1e-3
_K = 32


def _fps_body(npoint, px_ref, py_ref, pz_ref, nx_ref, ny_ref, nz_ref):
    px = px_ref[...]
    py = py_ref[...]
    pz = pz_ref[...]
    B, N = px.shape
    iota = jax.lax.broadcasted_iota(jnp.int32, (B, N), 1)
    col = jax.lax.broadcasted_iota(jnp.int32, (B, npoint), 1)

    def body(i, carry):
        dists, far, nx, ny, nz = carry
        mf = (iota == far).astype(jnp.float32)
        cx = jnp.sum(px * mf, axis=1, keepdims=True)
        cy = jnp.sum(py * mf, axis=1, keepdims=True)
        cz = jnp.sum(pz * mf, axis=1, keepdims=True)
        hit = col == i
        nx = jnp.where(hit, cx, nx)
        ny = jnp.where(hit, cy, ny)
        nz = jnp.where(hit, cz, nz)
        d = (px - cx) ** 2 + (py - cy) ** 2 + (pz - cz) ** 2
        dists = jnp.minimum(dists, d)
        mx = jnp.max(dists, axis=1, keepdims=True)
        far = jnp.min(jnp.where(dists == mx, iota, N), axis=1, keepdims=True)
        return dists, far.astype(jnp.int32), nx, ny, nz

    dists0 = jnp.full((B, N), 1e10, dtype=jnp.float32)
    far0 = jnp.zeros((B, 1), dtype=jnp.int32)
    z = jnp.zeros((B, npoint), dtype=jnp.float32)
    _, _, nx, ny, nz = jax.lax.fori_loop(0, npoint, body, (dists0, far0, z, z, z))
    nx_ref[...] = nx
    ny_ref[...] = ny
    nz_ref[...] = nz


def _fps(px, py, pz, npoint):
    B = px.shape[0]
    out = jax.ShapeDtypeStruct((B, npoint), jnp.float32)
    nx, ny, nz = pl.pallas_call(
        functools.partial(_fps_body, npoint),
        out_shape=[out, out, out],
    )(px, py, pz)
    return nx, ny, nz


def _rank_of_valid(valid_f, N):
    blk = min(N, 512)
    nb = N // blk
    i0 = jax.lax.broadcasted_iota(jnp.int32, (blk, blk), 0)
    i1 = jax.lax.broadcasted_iota(jnp.int32, (blk, blk), 1)
    U = (i0 < i1).astype(jnp.float32)
    Sc = valid_f.shape[0]
    offset = jnp.zeros((Sc, 1), dtype=jnp.float32)
    parts = []
    for j in range(nb):
        vj = valid_f[:, j * blk:(j + 1) * blk]
        parts.append(jnp.dot(vj, U, preferred_element_type=jnp.float32) + offset)
        offset = offset + jnp.sum(vj, axis=1, keepdims=True)
    return jnp.concatenate(parts, axis=1) if nb > 1 else parts[0]


def _stage_body(N, S, Sc, C, r, fuse_fa, nlayers, *refs):
    p3 = refs[0][0]
    px = refs[1][0]
    py = refs[2][0]
    pz = refs[3][0]
    q3 = refs[4][0]
    feats = refs[5][0]
    layer_refs = refs[6:6 + 3 * nlayers]
    pos = 6 + 3 * nlayers
    if fuse_fa:
        fa_refs = refs[pos:pos + 3]
        pos += 3
    out_ref = refs[pos]

    inv = 1.0 / math.sqrt(1.0 + _BN_EPS)
    qx = q3[:, 0:1]
    qy = q3[:, 1:2]
    qz = q3[:, 2:3]
    d = (qx - px) ** 2 + (qy - py) ** 2 + (qz - pz) ** 2
    valid = d <= r * r
    valid_f = valid.astype(jnp.float32)
    count = jnp.sum(valid_f, axis=1, keepdims=True)
    rank = _rank_of_valid(valid_f, N)
    rank_m = jnp.where(valid, rank, -1.0)

    pf = jnp.concatenate([p3, feats], axis=1)
    g_list = []
    g0 = None
    for k in range(_K):
        sel = (rank_m == float(k)).astype(jnp.float32)
        g = jnp.dot(sel, pf, preferred_element_type=jnp.float32)
        if k == 0:
            g0 = g
        else:
            g = jnp.where(count > float(k), g, g0)
        g_list.append(g)
    G = jnp.concatenate(g_list, axis=0)
    qt = jnp.concatenate([q3] * _K, axis=0)
    h = jnp.concatenate([(G[:, :3] - qt) / r, G[:, 3:]], axis=1)

    for li in range(nlayers):
        w = layer_refs[3 * li][...]
        gm = layer_refs[3 * li + 1][...]
        bt = layer_refs[3 * li + 2][...]
        h = jnp.dot(h, w, preferred_element_type=jnp.float32)
        h = jnp.maximum(h * (gm * inv) + bt, 0.0)

    C3 = h.shape[1]
    pooled = jnp.max(h.reshape(_K, Sc, C3), axis=0)

    if fuse_fa:
        wfa = fa_refs[0][...]
        gfa = fa_refs[1][...]
        bfa = fa_refs[2][...]
        cat = jnp.concatenate([q3, pooled], axis=1)
        hf = jnp.dot(cat, wfa, preferred_element_type=jnp.float32)
        hf = jnp.maximum(hf * (gfa * inv) + bfa, 0.0)
        out_ref[0, :, :] = jnp.max(hf, axis=0, keepdims=True)
    else:
        out_ref[0, :, :] = pooled


def _stage(p3, px, py, pz, q3, feats, layers, r, Sc, fa=None):
    B, N, _ = p3.shape
    S = q3.shape[1]
    C = feats.shape[2]
    nlayers = len(layers)
    nch = S // Sc
    fuse_fa = fa is not None

    args = [p3, px, py, pz, q3, feats]
    specs = [
        pl.BlockSpec((1, N, 3), lambda b, j: (b, 0, 0)),
        pl.BlockSpec((1, 1, N), lambda b, j: (b, 0, 0)),
        pl.BlockSpec((1, 1, N), lambda b, j: (b, 0, 0)),
        pl.BlockSpec((1, 1, N), lambda b, j: (b, 0, 0)),
        pl.BlockSpec((1, Sc, 3), lambda b, j: (b, j, 0)),
        pl.BlockSpec((1, N, C), lambda b, j: (b, 0, 0)),
    ]
    for lay in layers:
        wT = jnp.transpose(lay['w'])
        args += [wT, lay['gamma'][None, :], lay['beta'][None, :]]
        specs += [
            pl.BlockSpec(wT.shape, lambda b, j: (0, 0)),
            pl.BlockSpec((1, lay['w'].shape[0]), lambda b, j: (0, 0)),
            pl.BlockSpec((1, lay['w'].shape[0]), lambda b, j: (0, 0)),
        ]
    if fuse_fa:
        wfaT = jnp.transpose(fa['w'])
        args += [wfaT, fa['gamma'][None, :], fa['beta'][None, :]]
        specs += [
            pl.BlockSpec(wfaT.shape, lambda b, j: (0, 0)),
            pl.BlockSpec((1, 1024), lambda b, j: (0, 0)),
            pl.BlockSpec((1, 1024), lambda b, j: (0, 0)),
        ]
        cout = 1024
        out_shape = jax.ShapeDtypeStruct((B, nch, cout), jnp.float32)
        out_spec = pl.BlockSpec((1, 1, cout), lambda b, j: (b, j, 0))
    else:
        cout = layers[-1]['w'].shape[0]
        out_shape = jax.ShapeDtypeStruct((B, S, cout), jnp.float32)
        out_spec = pl.BlockSpec((1, Sc, cout), lambda b, j: (b, j, 0))

    body = functools.partial(_stage_body, N, S, Sc, C, r, fuse_fa, nlayers)
    return pl.pallas_call(
        body,
        grid=(B, nch),
        in_specs=specs,
        out_specs=out_spec,
        out_shape=out_shape,
    )(*args)


def kernel(points, params):
    xyz = points[..., :3]
    feats0 = points[..., 3:]
    px0 = xyz[..., 0]
    py0 = xyz[..., 1]
    pz0 = xyz[..., 2]

    nx1, ny1, nz1 = _fps(px0, py0, pz0, 512)
    q1 = jnp.stack([nx1, ny1, nz1], axis=-1)
    f1 = _stage(xyz, px0[:, None, :], py0[:, None, :], pz0[:, None, :],
                q1, feats0, params['sa0'], 0.3, 128)

    nx2, ny2, nz2 = _fps(nx1, ny1, nz1, 256)
    q2 = jnp.stack([nx2, ny2, nz2], axis=-1)
    f2 = _stage(q1, nx1[:, None, :], ny1[:, None, :], nz1[:, None, :],
                q2, f1, params['sa1'], 0.5, 128)

    nx3, ny3, nz3 = _fps(nx2, ny2, nz2, 128)
    q3 = jnp.stack([nx3, ny3, nz3], axis=-1)
    out = _stage(q2, nx2[:, None, :], ny2[:, None, :], nz2[:, None, :],
                 q3, f2, params['sa2'], 0.7, 128, fa=params['fa'])
    return jnp.transpose(out, (0, 2, 1))
